# baseline (speedup 1.0000x reference)
import jax
import jax.numpy as jnp
from jax import lax
from jax.experimental import pallas as pl
from jax.experimental.pallas import tpu as pltpu
from jax.experimental.pallas import tpu_sc as plsc

_B, _N = 64, 8192
_L = 16
_NW = 32
_RPW = _B // _NW
_NV = _N // _L


def _sc_body(x_hbm, kp1_hbm, tr_hbm, out_hbm, xv, keys, kp1v, trv, hist):
    wid = lax.axis_index("s") * 2 + lax.axis_index("c")
    base = wid * _RPW
    pltpu.sync_copy(x_hbm.at[pl.ds(base, _RPW)], xv)
    pltpu.sync_copy(kp1_hbm, kp1v)
    pltpu.sync_copy(tr_hbm, trv)
    train_nz = jnp.sum(trv[...]) != 0
    lanes = lax.iota(jnp.int32, _L)
    ones_v = jnp.ones((_L,), jnp.int32)

    for i in range(_RPW):
        r = base + i
        grp = r // _L
        lane = r % _L
        kv = kp1v[pl.ds(grp * _L, _L)]
        kp1 = jnp.sum(jnp.where(lanes == lane, kv, 0))

        def p0(j, acc):
            xk = xv[i, pl.ds(j * _L, _L)]
            b = plsc.bitcast(xk, jnp.int32)
            bu = plsc.bitcast(xk, jnp.uint32)
            keys[i, pl.ds(j * _L, _L)] = jnp.where(
                b < 0, ~bu, bu | jnp.uint32(0x80000000))
            return acc + xk
        num_acc = lax.fori_loop(0, _NV, p0, jnp.zeros((_L,), jnp.float32))
        num = jnp.sum(num_acc)

        prefix = jnp.zeros((_L,), jnp.uint32)
        target = jnp.zeros((_L,), jnp.int32) + kp1
        for s in range(4):
            shift = 24 - 8 * s
            himask = jnp.uint32((0xFFFFFFFF << (shift + 8)) & 0xFFFFFFFF)
            pm = prefix & himask

            def clr(g, _):
                z = jnp.zeros((_L,), jnp.int32)
                for l2 in range(_L):
                    hist[l2, pl.ds(g * _L, _L)] = z
                return 0
            lax.fori_loop(0, 16, clr, 0)

            def dp(j, _):
                key = keys[i, pl.ds(j * _L, _L)]
                match = (key & himask) == pm
                digit = plsc.bitcast(
                    (key >> jnp.uint32(shift)) & jnp.uint32(0xFF), jnp.int32)
                plsc.addupdate_scatter(hist, [lanes, digit], ones_v, mask=match)
                return 0
            lax.fori_loop(0, _NV, dp, 0)

            carry = jnp.zeros((_L,), jnp.int32)
            ltcnt = jnp.zeros((_L,), jnp.int32)
            cumbefore = jnp.zeros((_L,), jnp.int32)
            for g in range(16):
                tot = hist[0, pl.ds(g * _L, _L)]
                for l2 in range(1, _L):
                    tot = tot + hist[l2, pl.ds(g * _L, _L)]
                c = plsc.cumsum(tot) + carry
                carry = carry + jnp.sum(jnp.where(lanes == _L - 1, c - carry, 0))
                lt = c < target
                pc = plsc.all_reduce_population_count(lt)
                ltcnt = ltcnt + pc
                cb_g = jnp.sum(jnp.where(lt & (lanes == pc - 1), c, 0))
                cumbefore = jnp.where(pc > 0, cb_g, cumbefore)
            prefix = prefix | (plsc.bitcast(ltcnt, jnp.uint32)
                               << jnp.uint32(shift))
            target = target - cumbefore

        thr_bits = jnp.where(prefix >= jnp.uint32(0x80000000),
                             prefix ^ jnp.uint32(0x80000000), ~prefix)
        thr = plsc.bitcast(plsc.bitcast(thr_bits, jnp.int32), jnp.float32)

        def mp(j, acc):
            xk = xv[i, pl.ds(j * _L, _L)]
            mk = jnp.where((xk < thr) & train_nz, 0.0, xk)
            xv[i, pl.ds(j * _L, _L)] = mk
            return acc + mk
        den_acc = lax.fori_loop(0, _NV, mp, jnp.zeros((_L,), jnp.float32))
        num_v = jnp.zeros((_L,), jnp.float32) + num
        den_v = jnp.zeros((_L,), jnp.float32) + jnp.sum(den_acc)
        scale = jnp.abs(jnp.where(den_v == 0.0, 0.0, num_v / den_v))
        scale = jnp.where(train_nz, scale, 1.0)

        def sp(j, _):
            xv[i, pl.ds(j * _L, _L)] = xv[i, pl.ds(j * _L, _L)] * scale
            return 0
        lax.fori_loop(0, _NV, sp, 0)

    pltpu.sync_copy(xv, out_hbm.at[pl.ds(base, _RPW)])


def kernel(inputs, probs, training):
    B, N = inputs.shape
    idx = jnp.maximum(jnp.ceil(jnp.float32(N) * probs).astype(jnp.int32) - 1, 0)
    kp1 = idx + 1
    tr = jnp.full((_L,), (jnp.asarray(training) != 0).astype(jnp.int32))
    mesh = plsc.VectorSubcoreMesh(core_axis_name="c", subcore_axis_name="s")
    f = pl.kernel(
        _sc_body,
        mesh=mesh,
        compiler_params=pltpu.CompilerParams(needs_layout_passes=False),
        out_type=jax.ShapeDtypeStruct((B, N), jnp.float32),
        scratch_types=[
            pltpu.VMEM((_RPW, _N), jnp.float32),
            pltpu.VMEM((_RPW, _N), jnp.uint32),
            pltpu.VMEM((_B,), jnp.int32),
            pltpu.VMEM((_L,), jnp.int32),
            pltpu.VMEM((_L, 256), jnp.int32),
        ],
    )
    return f(inputs, kp1, tr)


# speedup vs baseline: 1.1707x; 1.1707x over previous
import jax
import jax.numpy as jnp
from jax import lax
from jax.experimental import pallas as pl
from jax.experimental.pallas import tpu as pltpu
from jax.experimental.pallas import tpu_sc as plsc

_B, _N = 64, 8192
_L = 16
_NW = 32
_RPW = _B // _NW
_NV = _N // _L


def _sc_body(x_hbm, kp1_hbm, tr_hbm, out_hbm, xv, keys, kp1v, trv, hist):
    wid = lax.axis_index("s") * 2 + lax.axis_index("c")
    base = wid * _RPW
    pltpu.sync_copy(x_hbm.at[pl.ds(base, _RPW)], xv)
    pltpu.sync_copy(kp1_hbm, kp1v)
    pltpu.sync_copy(tr_hbm, trv)
    train_nz = jnp.sum(trv[...]) != 0
    lanes = lax.iota(jnp.int32, _L)
    ones_v = jnp.ones((_L,), jnp.int32)

    def clear_hist():
        def clr(g, _):
            z = jnp.zeros((_L,), jnp.int32)
            for l2 in range(_L):
                hist[l2, pl.ds(g * _L, _L)] = z
            return 0
        lax.fori_loop(0, 16, clr, 0, unroll=4)

    def scan_hist(target):
        carry = jnp.zeros((_L,), jnp.int32)
        ltcnt = jnp.zeros((_L,), jnp.int32)
        cumbefore = jnp.zeros((_L,), jnp.int32)
        for g in range(16):
            tot = hist[0, pl.ds(g * _L, _L)]
            for l2 in range(1, _L):
                tot = tot + hist[l2, pl.ds(g * _L, _L)]
            c = plsc.cumsum(tot) + carry
            carry = carry + jnp.sum(jnp.where(lanes == _L - 1, c - carry, 0))
            lt = c < target
            pc = plsc.all_reduce_population_count(lt)
            ltcnt = ltcnt + pc
            cb_g = jnp.sum(jnp.where(lt & (lanes == pc - 1), c, 0))
            cumbefore = jnp.where(pc > 0, cb_g, cumbefore)
        return ltcnt, cumbefore

    for i in range(_RPW):
        r = base + i
        grp = r // _L
        lane = r % _L
        kv = kp1v[pl.ds(grp * _L, _L)]
        kp1 = jnp.sum(jnp.where(lanes == lane, kv, 0))

        clear_hist()

        def p0(j, acc):
            xk = xv[i, pl.ds(j * _L, _L)]
            b = plsc.bitcast(xk, jnp.int32)
            bu = plsc.bitcast(xk, jnp.uint32)
            key = jnp.where(b < 0, ~bu, bu | jnp.uint32(0x80000000))
            keys[i, pl.ds(j * _L, _L)] = key
            digit = plsc.bitcast(key >> jnp.uint32(24), jnp.int32)
            plsc.addupdate_scatter(hist, [lanes, digit], ones_v)
            return acc + xk
        num_acc = lax.fori_loop(0, _NV, p0, jnp.zeros((_L,), jnp.float32),
                                unroll=8)
        num = jnp.sum(num_acc)

        target = jnp.zeros((_L,), jnp.int32) + kp1
        b0, cb = scan_hist(target)
        prefix = plsc.bitcast(b0, jnp.uint32) << jnp.uint32(24)
        target = target - cb

        for s in range(1, 4):
            shift = 24 - 8 * s
            himask = jnp.uint32((0xFFFFFFFF << (shift + 8)) & 0xFFFFFFFF)
            pm = prefix & himask
            clear_hist()

            def dp(j, _):
                key = keys[i, pl.ds(j * _L, _L)]
                match = (key & himask) == pm
                digit = plsc.bitcast(
                    (key >> jnp.uint32(shift)) & jnp.uint32(0xFF), jnp.int32)
                plsc.addupdate_scatter(hist, [lanes, digit], ones_v, mask=match)
                return 0
            lax.fori_loop(0, _NV, dp, 0, unroll=8)

            bs, cb = scan_hist(target)
            prefix = prefix | (plsc.bitcast(bs, jnp.uint32)
                               << jnp.uint32(shift))
            target = target - cb

        thr_bits = jnp.where(prefix >= jnp.uint32(0x80000000),
                             prefix ^ jnp.uint32(0x80000000), ~prefix)
        thr = plsc.bitcast(plsc.bitcast(thr_bits, jnp.int32), jnp.float32)

        def mp(j, acc):
            xk = xv[i, pl.ds(j * _L, _L)]
            return acc + jnp.where(xk < thr, 0.0, xk)
        den_acc = lax.fori_loop(0, _NV, mp, jnp.zeros((_L,), jnp.float32),
                                unroll=8)
        num_v = jnp.zeros((_L,), jnp.float32) + num
        den_v = jnp.zeros((_L,), jnp.float32) + jnp.sum(den_acc)
        scale = jnp.abs(jnp.where(den_v == 0.0, 0.0, num_v / den_v))
        scale = jnp.where(train_nz, scale, 1.0)

        def sp(j, _):
            xk = xv[i, pl.ds(j * _L, _L)]
            mk = jnp.where((xk < thr) & train_nz, 0.0, xk)
            xv[i, pl.ds(j * _L, _L)] = mk * scale
            return 0
        lax.fori_loop(0, _NV, sp, 0, unroll=8)

    pltpu.sync_copy(xv, out_hbm.at[pl.ds(base, _RPW)])


def kernel(inputs, probs, training):
    B, N = inputs.shape
    idx = jnp.maximum(jnp.ceil(jnp.float32(N) * probs).astype(jnp.int32) - 1, 0)
    kp1 = idx + 1
    tr = jnp.full((_L,), (jnp.asarray(training) != 0).astype(jnp.int32))
    mesh = plsc.VectorSubcoreMesh(core_axis_name="c", subcore_axis_name="s")
    f = pl.kernel(
        _sc_body,
        mesh=mesh,
        compiler_params=pltpu.CompilerParams(needs_layout_passes=False),
        out_type=jax.ShapeDtypeStruct((B, N), jnp.float32),
        scratch_types=[
            pltpu.VMEM((_RPW, _N), jnp.float32),
            pltpu.VMEM((_RPW, _N), jnp.uint32),
            pltpu.VMEM((_B,), jnp.int32),
            pltpu.VMEM((_L,), jnp.int32),
            pltpu.VMEM((_L, 256), jnp.int32),
        ],
    )
    return f(inputs, kp1, tr)
